# baseline (speedup 1.0000x reference)
import functools

import jax
import jax.numpy as jnp
from jax.experimental import pallas as pl
from jax.experimental.pallas import tpu as pltpu

_LANE = 128
_N_BLK = 256
_G = 2


def _round_up(a, b):
    return (a + b - 1) // b * b


def _fused_kernel(n_classes, n_groups, w_img, b_ref, cbl_ref, x_ref, w1g_ref,
                  b1_ref, w2_ref, b2_ref, o_ref):
    k_g = b_ref.shape[0]
    acc = None
    for i in range(n_groups):
        sl = x_ref[:, _G * w_img * i:_G * w_img * i + k_g]
        zg = jnp.dot(sl, b_ref[...], preferred_element_type=jnp.float32)
        zg = jnp.maximum(zg + cbl_ref[...], 0.0)
        p = jnp.dot(zg, w1g_ref[i], preferred_element_type=jnp.float32)
        acc = p if acc is None else acc + p
    h = jnp.maximum(acc + b1_ref[...], 0.0)
    logits = jnp.dot(h, w2_ref[...], preferred_element_type=jnp.float32)
    logits = logits + b2_ref[...]
    col = jax.lax.broadcasted_iota(jnp.int32, logits.shape, 1)
    lm = jnp.where(col < n_classes, logits, -jnp.inf)
    m = jnp.max(lm, axis=1, keepdims=True)
    lse = jnp.log(jnp.sum(jnp.exp(lm - m), axis=1, keepdims=True)) + m
    o_ref[...] = (logits - lse)[:, :n_classes]


@jax.jit
def kernel(x, conv_w, conv_b, w1, b1, w2, b2):
    n, c_in, h_img, w_img = x.shape
    assert c_in == 1
    c_out = conv_w.shape[0]
    hid = w1.shape[1]
    n_classes = w2.shape[1]
    ho, wo = h_img - 2, w_img - 2
    n_groups = ho // _G
    k_g = (_G + 2) * w_img
    q_g = c_out * _G * wo

    x2 = x.reshape(n, h_img * w_img)
    n_blk = _N_BLK
    n_pad = _round_up(n, n_blk)
    if n_pad != n:
        x2 = jnp.pad(x2, ((0, n_pad - n), (0, 0)))

    hid_pad = _round_up(hid, _LANE)
    c_pad = _round_up(n_classes, _LANE)

    er = jnp.stack([jnp.eye(_G + 2, _G, -ky, dtype=jnp.float32)
                    for ky in range(3)])
    ex = jnp.stack([jnp.eye(w_img, wo, -kx, dtype=jnp.float32)
                    for kx in range(3)])
    bmat = jnp.einsum('ckl,krs,lxj->rxcsj', conv_w[:, 0], er, ex)
    bmat = bmat.reshape(k_g, q_g)
    cbl = jnp.repeat(conv_b, _G * wo).reshape(1, q_g)

    w1g = w1.reshape(c_out, n_groups, _G, wo, hid)
    w1g = jnp.transpose(w1g, (1, 0, 2, 3, 4)).reshape(n_groups, q_g, hid)
    w1g = jnp.pad(w1g, ((0, 0), (0, 0), (0, hid_pad - hid)))
    b1p = jnp.pad(b1, (0, hid_pad - hid)).reshape(1, hid_pad)
    w2p = jnp.pad(w2, ((0, hid_pad - hid), (0, c_pad - n_classes)))
    b2p = jnp.pad(b2, (0, c_pad - n_classes)).reshape(1, c_pad)

    grid = (n_pad // n_blk,)
    kern = functools.partial(_fused_kernel, n_classes, n_groups, w_img)
    out = pl.pallas_call(
        kern,
        out_shape=jax.ShapeDtypeStruct((n_pad, n_classes), jnp.float32),
        grid=grid,
        in_specs=[
            pl.BlockSpec((k_g, q_g), lambda i: (0, 0)),
            pl.BlockSpec((1, q_g), lambda i: (0, 0)),
            pl.BlockSpec((n_blk, h_img * w_img), lambda i: (i, 0)),
            pl.BlockSpec((n_groups, q_g, hid_pad), lambda i: (0, 0, 0)),
            pl.BlockSpec((1, hid_pad), lambda i: (0, 0)),
            pl.BlockSpec((hid_pad, c_pad), lambda i: (0, 0)),
            pl.BlockSpec((1, c_pad), lambda i: (0, 0)),
        ],
        out_specs=pl.BlockSpec((n_blk, n_classes), lambda i: (i, 0)),
        compiler_params=pltpu.CompilerParams(
            dimension_semantics=("parallel",)),
    )(bmat, cbl, x2, w1g, b1p, w2p, b2p)
    return out[:n]


# speedup vs baseline: 1.0692x; 1.0692x over previous
import functools

import jax
import jax.numpy as jnp
from jax.experimental import pallas as pl
from jax.experimental.pallas import tpu as pltpu

_LANE = 128
_N_BLK = 512
_G = 2


def _round_up(a, b):
    return (a + b - 1) // b * b


def _fused_kernel(n_classes, n_groups, w_img, b_ref, cbl_ref, x_ref, w1g_ref,
                  b1_ref, w2_ref, b2_ref, o_ref):
    k_g = b_ref.shape[0]
    acc = None
    for i in range(n_groups):
        sl = x_ref[:, _G * w_img * i:_G * w_img * i + k_g]
        zg = jnp.dot(sl, b_ref[...], preferred_element_type=jnp.float32)
        zg = jnp.maximum(zg + cbl_ref[...], 0.0)
        p = jnp.dot(zg, w1g_ref[i], preferred_element_type=jnp.float32)
        acc = p if acc is None else acc + p
    h = jnp.maximum(acc + b1_ref[...], 0.0)
    logits = jnp.dot(h, w2_ref[...], preferred_element_type=jnp.float32)
    logits = logits + b2_ref[...]
    col = jax.lax.broadcasted_iota(jnp.int32, logits.shape, 1)
    lm = jnp.where(col < n_classes, logits, -jnp.inf)
    m = jnp.max(lm, axis=1, keepdims=True)
    lse = jnp.log(jnp.sum(jnp.exp(lm - m), axis=1, keepdims=True)) + m
    o_ref[...] = (logits - lse)[:, :n_classes]


@jax.jit
def kernel(x, conv_w, conv_b, w1, b1, w2, b2):
    n, c_in, h_img, w_img = x.shape
    assert c_in == 1
    c_out = conv_w.shape[0]
    hid = w1.shape[1]
    n_classes = w2.shape[1]
    ho, wo = h_img - 2, w_img - 2
    n_groups = ho // _G
    k_g = (_G + 2) * w_img
    q_g = c_out * _G * wo

    x2 = x.reshape(n, h_img * w_img)
    n_blk = _N_BLK
    n_pad = _round_up(n, n_blk)
    if n_pad != n:
        x2 = jnp.pad(x2, ((0, n_pad - n), (0, 0)))

    hid_pad = _round_up(hid, _LANE)
    c_pad = _round_up(n_classes, _LANE)

    er = jnp.stack([jnp.eye(_G + 2, _G, -ky, dtype=jnp.float32)
                    for ky in range(3)])
    ex = jnp.stack([jnp.eye(w_img, wo, -kx, dtype=jnp.float32)
                    for kx in range(3)])
    bmat = jnp.einsum('ckl,krs,lxj->rxcsj', conv_w[:, 0], er, ex)
    bmat = bmat.reshape(k_g, q_g)
    cbl = jnp.repeat(conv_b, _G * wo).reshape(1, q_g)

    w1g = w1.reshape(c_out, n_groups, _G, wo, hid)
    w1g = jnp.transpose(w1g, (1, 0, 2, 3, 4)).reshape(n_groups, q_g, hid)
    w1g = jnp.pad(w1g, ((0, 0), (0, 0), (0, hid_pad - hid)))
    b1p = jnp.pad(b1, (0, hid_pad - hid)).reshape(1, hid_pad)
    w2p = jnp.pad(w2, ((0, hid_pad - hid), (0, c_pad - n_classes)))
    b2p = jnp.pad(b2, (0, c_pad - n_classes)).reshape(1, c_pad)

    grid = (n_pad // n_blk,)
    kern = functools.partial(_fused_kernel, n_classes, n_groups, w_img)
    out = pl.pallas_call(
        kern,
        out_shape=jax.ShapeDtypeStruct((n_pad, n_classes), jnp.float32),
        grid=grid,
        in_specs=[
            pl.BlockSpec((k_g, q_g), lambda i: (0, 0)),
            pl.BlockSpec((1, q_g), lambda i: (0, 0)),
            pl.BlockSpec((n_blk, h_img * w_img), lambda i: (i, 0)),
            pl.BlockSpec((n_groups, q_g, hid_pad), lambda i: (0, 0, 0)),
            pl.BlockSpec((1, hid_pad), lambda i: (0, 0)),
            pl.BlockSpec((hid_pad, c_pad), lambda i: (0, 0)),
            pl.BlockSpec((1, c_pad), lambda i: (0, 0)),
        ],
        out_specs=pl.BlockSpec((n_blk, n_classes), lambda i: (i, 0)),
        compiler_params=pltpu.CompilerParams(
            dimension_semantics=("parallel",)),
    )(bmat, cbl, x2, w1g, b1p, w2p, b2p)
    return out[:n]


# speedup vs baseline: 1.1009x; 1.0297x over previous
import functools

import jax
import jax.numpy as jnp
from jax.experimental import pallas as pl
from jax.experimental.pallas import tpu as pltpu

_LANE = 128
_N_BLK = 1024
_G = 2


def _round_up(a, b):
    return (a + b - 1) // b * b


def _fused_kernel(n_classes, n_groups, w_img, b_ref, cbl_ref, x_ref, w1g_ref,
                  b1_ref, w2_ref, b2_ref, o_ref):
    k_g = b_ref.shape[0]
    acc = None
    for i in range(n_groups):
        sl = x_ref[:, _G * w_img * i:_G * w_img * i + k_g]
        zg = jnp.dot(sl, b_ref[...], preferred_element_type=jnp.float32)
        zg = jnp.maximum(zg + cbl_ref[...], 0.0)
        p = jnp.dot(zg, w1g_ref[i], preferred_element_type=jnp.float32)
        acc = p if acc is None else acc + p
    h = jnp.maximum(acc + b1_ref[...], 0.0)
    logits = jnp.dot(h, w2_ref[...], preferred_element_type=jnp.float32)
    logits = logits + b2_ref[...]
    col = jax.lax.broadcasted_iota(jnp.int32, logits.shape, 1)
    lm = jnp.where(col < n_classes, logits, -jnp.inf)
    m = jnp.max(lm, axis=1, keepdims=True)
    lse = jnp.log(jnp.sum(jnp.exp(lm - m), axis=1, keepdims=True)) + m
    o_ref[...] = (logits - lse)[:, :n_classes]


@jax.jit
def kernel(x, conv_w, conv_b, w1, b1, w2, b2):
    n, c_in, h_img, w_img = x.shape
    assert c_in == 1
    c_out = conv_w.shape[0]
    hid = w1.shape[1]
    n_classes = w2.shape[1]
    ho, wo = h_img - 2, w_img - 2
    n_groups = ho // _G
    k_g = (_G + 2) * w_img
    q_g = c_out * _G * wo

    x2 = x.reshape(n, h_img * w_img)
    n_blk = _N_BLK
    n_pad = _round_up(n, n_blk)
    if n_pad != n:
        x2 = jnp.pad(x2, ((0, n_pad - n), (0, 0)))

    hid_pad = _round_up(hid, _LANE)
    c_pad = _round_up(n_classes, _LANE)

    er = jnp.stack([jnp.eye(_G + 2, _G, -ky, dtype=jnp.float32)
                    for ky in range(3)])
    ex = jnp.stack([jnp.eye(w_img, wo, -kx, dtype=jnp.float32)
                    for kx in range(3)])
    bmat = jnp.einsum('ckl,krs,lxj->rxcsj', conv_w[:, 0], er, ex)
    bmat = bmat.reshape(k_g, q_g)
    cbl = jnp.repeat(conv_b, _G * wo).reshape(1, q_g)

    w1g = w1.reshape(c_out, n_groups, _G, wo, hid)
    w1g = jnp.transpose(w1g, (1, 0, 2, 3, 4)).reshape(n_groups, q_g, hid)
    w1g = jnp.pad(w1g, ((0, 0), (0, 0), (0, hid_pad - hid)))
    b1p = jnp.pad(b1, (0, hid_pad - hid)).reshape(1, hid_pad)
    w2p = jnp.pad(w2, ((0, hid_pad - hid), (0, c_pad - n_classes)))
    b2p = jnp.pad(b2, (0, c_pad - n_classes)).reshape(1, c_pad)

    grid = (n_pad // n_blk,)
    kern = functools.partial(_fused_kernel, n_classes, n_groups, w_img)
    out = pl.pallas_call(
        kern,
        out_shape=jax.ShapeDtypeStruct((n_pad, n_classes), jnp.float32),
        grid=grid,
        in_specs=[
            pl.BlockSpec((k_g, q_g), lambda i: (0, 0)),
            pl.BlockSpec((1, q_g), lambda i: (0, 0)),
            pl.BlockSpec((n_blk, h_img * w_img), lambda i: (i, 0)),
            pl.BlockSpec((n_groups, q_g, hid_pad), lambda i: (0, 0, 0)),
            pl.BlockSpec((1, hid_pad), lambda i: (0, 0)),
            pl.BlockSpec((hid_pad, c_pad), lambda i: (0, 0)),
            pl.BlockSpec((1, c_pad), lambda i: (0, 0)),
        ],
        out_specs=pl.BlockSpec((n_blk, n_classes), lambda i: (i, 0)),
        compiler_params=pltpu.CompilerParams(
            dimension_semantics=("parallel",)),
    )(bmat, cbl, x2, w1g, b1p, w2p, b2p)
    return out[:n]


# speedup vs baseline: 1.1010x; 1.0001x over previous
import functools

import jax
import jax.numpy as jnp
from jax.experimental import pallas as pl
from jax.experimental.pallas import tpu as pltpu

_LANE = 128
_N_BLK = 2048
_G = 2


def _round_up(a, b):
    return (a + b - 1) // b * b


def _fused_kernel(n_classes, n_groups, w_img, b_ref, cbl_ref, x_ref, w1g_ref,
                  b1_ref, w2_ref, b2_ref, o_ref):
    k_g = b_ref.shape[0]
    acc = None
    for i in range(n_groups):
        sl = x_ref[:, _G * w_img * i:_G * w_img * i + k_g]
        zg = jnp.dot(sl, b_ref[...], preferred_element_type=jnp.float32)
        zg = jnp.maximum(zg + cbl_ref[...], 0.0)
        p = jnp.dot(zg, w1g_ref[i], preferred_element_type=jnp.float32)
        acc = p if acc is None else acc + p
    h = jnp.maximum(acc + b1_ref[...], 0.0)
    logits = jnp.dot(h, w2_ref[...], preferred_element_type=jnp.float32)
    logits = logits + b2_ref[...]
    col = jax.lax.broadcasted_iota(jnp.int32, logits.shape, 1)
    lm = jnp.where(col < n_classes, logits, -jnp.inf)
    m = jnp.max(lm, axis=1, keepdims=True)
    lse = jnp.log(jnp.sum(jnp.exp(lm - m), axis=1, keepdims=True)) + m
    o_ref[...] = (logits - lse)[:, :n_classes]


@jax.jit
def kernel(x, conv_w, conv_b, w1, b1, w2, b2):
    n, c_in, h_img, w_img = x.shape
    assert c_in == 1
    c_out = conv_w.shape[0]
    hid = w1.shape[1]
    n_classes = w2.shape[1]
    ho, wo = h_img - 2, w_img - 2
    n_groups = ho // _G
    k_g = (_G + 2) * w_img
    q_g = c_out * _G * wo

    x2 = x.reshape(n, h_img * w_img)
    n_blk = _N_BLK
    n_pad = _round_up(n, n_blk)
    if n_pad != n:
        x2 = jnp.pad(x2, ((0, n_pad - n), (0, 0)))

    hid_pad = _round_up(hid, _LANE)
    c_pad = _round_up(n_classes, _LANE)

    er = jnp.stack([jnp.eye(_G + 2, _G, -ky, dtype=jnp.float32)
                    for ky in range(3)])
    ex = jnp.stack([jnp.eye(w_img, wo, -kx, dtype=jnp.float32)
                    for kx in range(3)])
    bmat = jnp.einsum('ckl,krs,lxj->rxcsj', conv_w[:, 0], er, ex)
    bmat = bmat.reshape(k_g, q_g)
    cbl = jnp.repeat(conv_b, _G * wo).reshape(1, q_g)

    w1g = w1.reshape(c_out, n_groups, _G, wo, hid)
    w1g = jnp.transpose(w1g, (1, 0, 2, 3, 4)).reshape(n_groups, q_g, hid)
    w1g = jnp.pad(w1g, ((0, 0), (0, 0), (0, hid_pad - hid)))
    b1p = jnp.pad(b1, (0, hid_pad - hid)).reshape(1, hid_pad)
    w2p = jnp.pad(w2, ((0, hid_pad - hid), (0, c_pad - n_classes)))
    b2p = jnp.pad(b2, (0, c_pad - n_classes)).reshape(1, c_pad)

    grid = (n_pad // n_blk,)
    kern = functools.partial(_fused_kernel, n_classes, n_groups, w_img)
    out = pl.pallas_call(
        kern,
        out_shape=jax.ShapeDtypeStruct((n_pad, n_classes), jnp.float32),
        grid=grid,
        in_specs=[
            pl.BlockSpec((k_g, q_g), lambda i: (0, 0)),
            pl.BlockSpec((1, q_g), lambda i: (0, 0)),
            pl.BlockSpec((n_blk, h_img * w_img), lambda i: (i, 0)),
            pl.BlockSpec((n_groups, q_g, hid_pad), lambda i: (0, 0, 0)),
            pl.BlockSpec((1, hid_pad), lambda i: (0, 0)),
            pl.BlockSpec((hid_pad, c_pad), lambda i: (0, 0)),
            pl.BlockSpec((1, c_pad), lambda i: (0, 0)),
        ],
        out_specs=pl.BlockSpec((n_blk, n_classes), lambda i: (i, 0)),
        compiler_params=pltpu.CompilerParams(
            dimension_semantics=("parallel",)),
    )(bmat, cbl, x2, w1g, b1p, w2p, b2p)
    return out[:n]


# speedup vs baseline: 1.1616x; 1.0551x over previous
import functools

import jax
import jax.numpy as jnp
from jax.experimental import pallas as pl
from jax.experimental.pallas import tpu as pltpu

_LANE = 128
_N_BLK = 1024
_G = 2


def _round_up(a, b):
    return (a + b - 1) // b * b


def _fused_kernel(n_classes, n_groups, w_img, b_ref, cbl_ref, x_ref, w1g_ref,
                  b1_ref, w2_ref, b2_ref, o_ref):
    k_g = b_ref.shape[0]
    acc = None
    for i in range(n_groups):
        sl = x_ref[:, _G * w_img * i:_G * w_img * i + k_g]
        zg = jnp.dot(sl, b_ref[...], preferred_element_type=jnp.float32)
        zg = jnp.maximum(zg + cbl_ref[...], 0.0)
        p = jnp.dot(zg, w1g_ref[i], preferred_element_type=jnp.float32)
        acc = p if acc is None else acc + p
    h = jnp.maximum(acc + b1_ref[...], 0.0)
    logits = jnp.dot(h, w2_ref[...], preferred_element_type=jnp.float32)
    logits = logits + b2_ref[...]
    col = jax.lax.broadcasted_iota(jnp.int32, logits.shape, 1)
    lm = jnp.where(col < n_classes, logits, -jnp.inf)
    m = jnp.max(lm, axis=1, keepdims=True)
    lse = jnp.log(jnp.sum(jnp.exp(lm - m), axis=1, keepdims=True)) + m
    o_ref[...] = (logits - lse)[:, :n_classes]


@jax.jit
def kernel(x, conv_w, conv_b, w1, b1, w2, b2):
    n, c_in, h_img, w_img = x.shape
    assert c_in == 1
    c_out = conv_w.shape[0]
    hid = w1.shape[1]
    n_classes = w2.shape[1]
    ho, wo = h_img - 2, w_img - 2
    n_groups = ho // _G
    k_g = (_G + 2) * w_img
    q_g = c_out * _G * wo

    x2 = x.reshape(n, h_img * w_img).astype(jnp.bfloat16)
    n_blk = _N_BLK
    n_pad = _round_up(n, n_blk)
    if n_pad != n:
        x2 = jnp.pad(x2, ((0, n_pad - n), (0, 0)))

    hid_pad = _round_up(hid, _LANE)
    c_pad = _round_up(n_classes, _LANE)

    er = jnp.stack([jnp.eye(_G + 2, _G, -ky, dtype=jnp.float32)
                    for ky in range(3)])
    ex = jnp.stack([jnp.eye(w_img, wo, -kx, dtype=jnp.float32)
                    for kx in range(3)])
    bmat = jnp.einsum('ckl,krs,lxj->rxcsj', conv_w[:, 0], er, ex)
    bmat = bmat.reshape(k_g, q_g).astype(jnp.bfloat16)
    cbl = jnp.repeat(conv_b, _G * wo).reshape(1, q_g)

    w1g = w1.reshape(c_out, n_groups, _G, wo, hid)
    w1g = jnp.transpose(w1g, (1, 0, 2, 3, 4)).reshape(n_groups, q_g, hid)
    w1g = jnp.pad(w1g, ((0, 0), (0, 0), (0, hid_pad - hid)))
    b1p = jnp.pad(b1, (0, hid_pad - hid)).reshape(1, hid_pad)
    w2p = jnp.pad(w2, ((0, hid_pad - hid), (0, c_pad - n_classes)))
    b2p = jnp.pad(b2, (0, c_pad - n_classes)).reshape(1, c_pad)

    grid = (n_pad // n_blk,)
    kern = functools.partial(_fused_kernel, n_classes, n_groups, w_img)
    out = pl.pallas_call(
        kern,
        out_shape=jax.ShapeDtypeStruct((n_pad, n_classes), jnp.float32),
        grid=grid,
        in_specs=[
            pl.BlockSpec((k_g, q_g), lambda i: (0, 0)),
            pl.BlockSpec((1, q_g), lambda i: (0, 0)),
            pl.BlockSpec((n_blk, h_img * w_img), lambda i: (i, 0)),
            pl.BlockSpec((n_groups, q_g, hid_pad), lambda i: (0, 0, 0)),
            pl.BlockSpec((1, hid_pad), lambda i: (0, 0)),
            pl.BlockSpec((hid_pad, c_pad), lambda i: (0, 0)),
            pl.BlockSpec((1, c_pad), lambda i: (0, 0)),
        ],
        out_specs=pl.BlockSpec((n_blk, n_classes), lambda i: (i, 0)),
        compiler_params=pltpu.CompilerParams(
            dimension_semantics=("parallel",)),
    )(bmat, cbl, x2, w1g, b1p, w2p, b2p)
    return out[:n]


# speedup vs baseline: 1.1656x; 1.0034x over previous
import functools

import jax
import jax.numpy as jnp
from jax.experimental import pallas as pl
from jax.experimental.pallas import tpu as pltpu

_LANE = 128
_N_BLK = 1024
_G = 2


def _round_up(a, b):
    return (a + b - 1) // b * b


def _fused_kernel(n_classes, n_groups, w_img, b_ref, cbl_ref, x_ref, w1g_ref,
                  b1_ref, w2_ref, b2_ref, o_ref):
    k_g = b_ref.shape[0]
    acc = None
    for i in range(n_groups):
        sl = x_ref[:, _G * w_img * i:_G * w_img * i + k_g]
        zg = jnp.dot(sl, b_ref[...], preferred_element_type=jnp.float32)
        zg = jnp.maximum(zg + cbl_ref[...], 0.0)
        p = jnp.dot(zg.astype(jnp.bfloat16), w1g_ref[i],
                    preferred_element_type=jnp.float32)
        acc = p if acc is None else acc + p
    h = jnp.maximum(acc + b1_ref[...], 0.0)
    logits = jnp.dot(h, w2_ref[...], preferred_element_type=jnp.float32)
    logits = logits + b2_ref[...]
    col = jax.lax.broadcasted_iota(jnp.int32, logits.shape, 1)
    lm = jnp.where(col < n_classes, logits, -jnp.inf)
    m = jnp.max(lm, axis=1, keepdims=True)
    lse = jnp.log(jnp.sum(jnp.exp(lm - m), axis=1, keepdims=True)) + m
    o_ref[...] = (logits - lse)[:, :n_classes]


@jax.jit
def kernel(x, conv_w, conv_b, w1, b1, w2, b2):
    n, c_in, h_img, w_img = x.shape
    assert c_in == 1
    c_out = conv_w.shape[0]
    hid = w1.shape[1]
    n_classes = w2.shape[1]
    ho, wo = h_img - 2, w_img - 2
    n_groups = ho // _G
    k_g = (_G + 2) * w_img
    q_g = c_out * _G * wo

    x2 = x.reshape(n, h_img * w_img).astype(jnp.bfloat16)
    n_blk = _N_BLK
    n_pad = _round_up(n, n_blk)
    if n_pad != n:
        x2 = jnp.pad(x2, ((0, n_pad - n), (0, 0)))

    hid_pad = _round_up(hid, _LANE)
    c_pad = _round_up(n_classes, _LANE)

    er = jnp.stack([jnp.eye(_G + 2, _G, -ky, dtype=jnp.float32)
                    for ky in range(3)])
    ex = jnp.stack([jnp.eye(w_img, wo, -kx, dtype=jnp.float32)
                    for kx in range(3)])
    bmat = jnp.einsum('ckl,krs,lxj->rxcsj', conv_w[:, 0], er, ex)
    bmat = bmat.reshape(k_g, q_g).astype(jnp.bfloat16)
    cbl = jnp.repeat(conv_b, _G * wo).reshape(1, q_g)

    w1g = w1.reshape(c_out, n_groups, _G, wo, hid)
    w1g = jnp.transpose(w1g, (1, 0, 2, 3, 4)).reshape(n_groups, q_g, hid)
    w1g = jnp.pad(w1g, ((0, 0), (0, 0), (0, hid_pad - hid)))
    w1g = w1g.astype(jnp.bfloat16)
    b1p = jnp.pad(b1, (0, hid_pad - hid)).reshape(1, hid_pad)
    w2p = jnp.pad(w2, ((0, hid_pad - hid), (0, c_pad - n_classes)))
    b2p = jnp.pad(b2, (0, c_pad - n_classes)).reshape(1, c_pad)

    grid = (n_pad // n_blk,)
    kern = functools.partial(_fused_kernel, n_classes, n_groups, w_img)
    out = pl.pallas_call(
        kern,
        out_shape=jax.ShapeDtypeStruct((n_pad, n_classes), jnp.float32),
        grid=grid,
        in_specs=[
            pl.BlockSpec((k_g, q_g), lambda i: (0, 0)),
            pl.BlockSpec((1, q_g), lambda i: (0, 0)),
            pl.BlockSpec((n_blk, h_img * w_img), lambda i: (i, 0)),
            pl.BlockSpec((n_groups, q_g, hid_pad), lambda i: (0, 0, 0)),
            pl.BlockSpec((1, hid_pad), lambda i: (0, 0)),
            pl.BlockSpec((hid_pad, c_pad), lambda i: (0, 0)),
            pl.BlockSpec((1, c_pad), lambda i: (0, 0)),
        ],
        out_specs=pl.BlockSpec((n_blk, n_classes), lambda i: (i, 0)),
        compiler_params=pltpu.CompilerParams(
            dimension_semantics=("parallel",)),
    )(bmat, cbl, x2, w1g, b1p, w2p, b2p)
    return out[:n]
